# Initial kernel scaffold; baseline (speedup 1.0000x reference)
import jax
import jax.numpy as jnp
from jax.experimental import pallas as pl


def kernel(hidden_state, W_in, b_in, codebook, W_out, b_out):
    raise NotImplementedError("write your pallas kernel here")


# speedup vs baseline: 3.3311x; 3.3311x over previous
import functools

import jax
import jax.numpy as jnp
from jax.experimental import pallas as pl
from jax.experimental.pallas import tpu as pltpu

B, H, T = 8, 1024, 4096
CD, CS = 8, 1024
TT = 1024


def _vq_kernel(h_ref, w_in_ref, b_in_ref, cb_ref, w_out_ref, b_out_ref,
               out_ref, idx_ref, proj_ref, sse_ref):
    b = pl.program_id(0)
    t = pl.program_id(1)

    h = h_ref[0]
    w_in = w_in_ref[...]
    cb = cb_ref[...]
    w_out = w_out_ref[...]

    p = jax.lax.dot_general(w_in, h, (((1,), (0,)), ((), ())),
                            preferred_element_type=jnp.float32)
    p = p + b_in_ref[...][:, None]
    proj_ref[0] = p

    n = jnp.sqrt(jnp.sum(p * p, axis=0, keepdims=True))
    en = p / jnp.maximum(n, 1e-12)
    cbn = jnp.sqrt(jnp.sum(cb * cb, axis=1, keepdims=True))
    cn = cb / jnp.maximum(cbn, 1e-12)

    l2 = jnp.sum(en * en, axis=0, keepdims=True)
    cn2 = jnp.sum(cn * cn, axis=1, keepdims=True)
    sc = jax.lax.dot_general(cn, en, (((1,), (0,)), ((), ())),
                             preferred_element_type=jnp.float32)
    dist = -(l2 - 2.0 * sc) + cn2

    idx = jnp.argmax(dist, axis=0).astype(jnp.int32)
    idx_ref[0, 0] = idx

    iota = jax.lax.broadcasted_iota(jnp.int32, (CS, TT), 0)
    onehot = (iota == idx[None, :]).astype(jnp.float32)
    q = jax.lax.dot_general(cb, onehot, (((0,), (0,)), ((), ())),
                            preferred_element_type=jnp.float32)

    d = p - q
    sse = jnp.sum(d * d)

    @pl.when(jnp.logical_and(b == 0, t == 0))
    def _init():
        sse_ref[0, 0] = 0.0

    sse_ref[0, 0] += sse

    o = jax.lax.dot_general(w_out, q, (((1,), (0,)), ((), ())),
                            preferred_element_type=jnp.float32)
    out_ref[0] = o + b_out_ref[...][:, None]


@functools.partial(jax.jit, static_argnames=("interpret",))
def _vq(hidden_state, W_in, b_in, codebook, W_out, b_out, interpret=False):
    grid = (B, T // TT)
    out, idx3, proj, sse = pl.pallas_call(
        _vq_kernel,
        grid=grid,
        in_specs=[
            pl.BlockSpec((1, H, TT), lambda b, t: (b, 0, t)),
            pl.BlockSpec((CD, H), lambda b, t: (0, 0)),
            pl.BlockSpec((CD,), lambda b, t: (0,)),
            pl.BlockSpec((CS, CD), lambda b, t: (0, 0)),
            pl.BlockSpec((H, CD), lambda b, t: (0, 0)),
            pl.BlockSpec((H,), lambda b, t: (0,)),
        ],
        out_specs=[
            pl.BlockSpec((1, H, TT), lambda b, t: (b, 0, t)),
            pl.BlockSpec((1, 1, TT), lambda b, t: (b, 0, t)),
            pl.BlockSpec((1, CD, TT), lambda b, t: (b, 0, t)),
            pl.BlockSpec(memory_space=pltpu.SMEM, block_shape=(1, 1),
                         index_map=lambda b, t: (0, 0)),
        ],
        out_shape=[
            jax.ShapeDtypeStruct((B, H, T), jnp.float32),
            jax.ShapeDtypeStruct((B, 1, T), jnp.int32),
            jax.ShapeDtypeStruct((B, CD, T), jnp.float32),
            jax.ShapeDtypeStruct((1, 1), jnp.float32),
        ],
        interpret=interpret,
    )(hidden_state, W_in, b_in, codebook, W_out, b_out)
    loss = sse[0, 0] / (B * CD * T)
    return out, loss, loss, idx3.reshape(B, T), proj


def kernel(hidden_state, W_in, b_in, codebook, W_out, b_out):
    return _vq(hidden_state, W_in, b_in, codebook, W_out, b_out)


# speedup vs baseline: 3.6886x; 1.1073x over previous
import functools

import jax
import jax.numpy as jnp
from jax.experimental import pallas as pl
from jax.experimental.pallas import tpu as pltpu

B, H, T = 8, 1024, 4096
CD, CS = 8, 1024
TT = 2048


def _vq_kernel(h_ref, w_in_ref, b_in_ref, cb_ref, w_out_ref, b_out_ref,
               out_ref, idx_ref, proj_ref, sse_ref):
    b = pl.program_id(0)
    t = pl.program_id(1)

    h = h_ref[0]
    w_in = w_in_ref[...]
    cb = cb_ref[...]
    w_out = w_out_ref[...]

    p = jax.lax.dot_general(w_in, h, (((1,), (0,)), ((), ())),
                            preferred_element_type=jnp.float32)
    p = p + b_in_ref[...][:, None]
    proj_ref[0] = p

    n = jnp.sqrt(jnp.sum(p * p, axis=0, keepdims=True))
    en = p / jnp.maximum(n, 1e-12)
    cbn = jnp.sqrt(jnp.sum(cb * cb, axis=1, keepdims=True))
    cn = cb / jnp.maximum(cbn, 1e-12)

    l2 = jnp.sum(en * en, axis=0, keepdims=True)
    cn2 = jnp.sum(cn * cn, axis=1, keepdims=True)
    sc = jax.lax.dot_general(cn, en, (((1,), (0,)), ((), ())),
                             preferred_element_type=jnp.float32)
    dist = -(l2 - 2.0 * sc) + cn2

    idx = jnp.argmax(dist, axis=0).astype(jnp.int32)
    idx_ref[0, 0] = idx

    iota = jax.lax.broadcasted_iota(jnp.int32, (CS, TT), 0)
    onehot = (iota == idx[None, :]).astype(jnp.float32)
    q = jax.lax.dot_general(cb, onehot, (((0,), (0,)), ((), ())),
                            preferred_element_type=jnp.float32)

    d = p - q
    sse = jnp.sum(d * d)

    @pl.when(jnp.logical_and(b == 0, t == 0))
    def _init():
        sse_ref[0, 0] = 0.0

    sse_ref[0, 0] += sse

    o = jax.lax.dot_general(w_out, q, (((1,), (0,)), ((), ())),
                            preferred_element_type=jnp.float32)
    out_ref[0] = o + b_out_ref[...][:, None]


@functools.partial(jax.jit, static_argnames=("interpret",))
def _vq(hidden_state, W_in, b_in, codebook, W_out, b_out, interpret=False):
    grid = (B, T // TT)
    out, idx3, proj, sse = pl.pallas_call(
        _vq_kernel,
        grid=grid,
        in_specs=[
            pl.BlockSpec((1, H, TT), lambda b, t: (b, 0, t)),
            pl.BlockSpec((CD, H), lambda b, t: (0, 0)),
            pl.BlockSpec((CD,), lambda b, t: (0,)),
            pl.BlockSpec((CS, CD), lambda b, t: (0, 0)),
            pl.BlockSpec((H, CD), lambda b, t: (0, 0)),
            pl.BlockSpec((H,), lambda b, t: (0,)),
        ],
        out_specs=[
            pl.BlockSpec((1, H, TT), lambda b, t: (b, 0, t)),
            pl.BlockSpec((1, 1, TT), lambda b, t: (b, 0, t)),
            pl.BlockSpec((1, CD, TT), lambda b, t: (b, 0, t)),
            pl.BlockSpec(memory_space=pltpu.SMEM, block_shape=(1, 1),
                         index_map=lambda b, t: (0, 0)),
        ],
        out_shape=[
            jax.ShapeDtypeStruct((B, H, T), jnp.float32),
            jax.ShapeDtypeStruct((B, 1, T), jnp.int32),
            jax.ShapeDtypeStruct((B, CD, T), jnp.float32),
            jax.ShapeDtypeStruct((1, 1), jnp.float32),
        ],
        interpret=interpret,
    )(hidden_state, W_in, b_in, codebook, W_out, b_out)
    loss = sse[0, 0] / (B * CD * T)
    return out, loss, loss, idx3.reshape(B, T), proj


def kernel(hidden_state, W_in, b_in, codebook, W_out, b_out):
    return _vq(hidden_state, W_in, b_in, codebook, W_out, b_out)
